# baseline (speedup 1.0000x reference)
import jax
import jax.numpy as jnp
from jax.experimental import pallas as pl
from jax.experimental.pallas import tpu as pltpu

_PACK = 4
_BM = 256
_NSLOT = 4
_NQ = 4


def _body(cw_ref, cb_ref, g_ref, b2_ref, x_hbm, o_hbm, xbuf, obuf, insem, outsem):
    Bp = x_hbm.shape[0]
    nsteps = Bp // _BM
    qrows = Bp // _NQ

    def in_copy(q):
        return pltpu.make_async_copy(
            x_hbm.at[pl.ds(q * qrows, qrows)],
            xbuf.at[pl.ds(q * qrows, qrows)],
            insem.at[q])

    def out_copy(i, slot):
        return pltpu.make_async_copy(
            obuf.at[slot], o_hbm.at[pl.ds(i * _BM, _BM)], outsem.at[slot])

    for q in range(_NQ):
        in_copy(q).start()

    steps_per_q = nsteps // _NQ

    def loop(i, carry):
        slot = jax.lax.rem(i, _NSLOT)
        @pl.when(jax.lax.rem(i, steps_per_q) == 0)
        def _():
            in_copy(jax.lax.div(i, steps_per_q)).wait()
        u = jnp.maximum(
            xbuf[pl.ds(i * _BM, _BM)] * cw_ref[...] + cb_ref[...], 0.0)
        acc = jnp.dot(u, g_ref[...], preferred_element_type=jnp.float32)
        @pl.when(i >= _NSLOT)
        def _():
            out_copy(i - _NSLOT, slot).wait()
        obuf[slot] = jnp.maximum(acc + b2_ref[...], 0.0)
        out_copy(i, slot).start()
        return carry

    jax.lax.fori_loop(0, nsteps, loop, 0)
    for j in range(max(0, nsteps - _NSLOT), nsteps):
        out_copy(j, j % _NSLOT).wait()


def kernel(x, conv_w, conv_b, W0, b0, W1, b1, W2, b2):
    B, I = x.shape
    N = W2.shape[0]
    K = W2.shape[1] // 2
    P = _PACK
    G = jnp.zeros((I, N), x.dtype).at[33::2, :].set(W2[:, :K].T)
    G4 = jnp.zeros((P, I, P, N), x.dtype)
    for r in range(P):
        G4 = G4.at[r, :, r, :].set(G)
    G4 = G4.reshape(P * I, P * N)
    cw4 = jnp.tile(conv_w, P)[None]
    cb4 = jnp.tile(conv_b, P)[None]
    b24 = jnp.tile(b2, P)[None]
    xr = x.reshape(B // P, P * I)
    vmem = pl.BlockSpec(memory_space=pltpu.VMEM)
    hbm = pl.BlockSpec(memory_space=pl.ANY)
    out = pl.pallas_call(
        _body,
        in_specs=[vmem, vmem, vmem, vmem, hbm],
        out_specs=hbm,
        out_shape=jax.ShapeDtypeStruct((B // P, P * N), x.dtype),
        scratch_shapes=[
            pltpu.VMEM((B // P, P * I), x.dtype),
            pltpu.VMEM((_NSLOT, _BM, P * N), x.dtype),
            pltpu.SemaphoreType.DMA((_NQ,)),
            pltpu.SemaphoreType.DMA((_NSLOT,)),
        ],
    )(cw4, cb4, G4, b24, xr)
    return out.reshape(B, N)


# speedup vs baseline: 1.0522x; 1.0522x over previous
import jax
import jax.numpy as jnp
from jax.experimental import pallas as pl
from jax.experimental.pallas import tpu as pltpu

_BM = 512
_NSLOT = 4
_NQ = 4


def _body(cw_ref, cb_ref, g_ref, b2_ref, x_hbm, o_hbm, xbuf, obuf, insem, outsem):
    Bp = x_hbm.shape[0]
    nsteps = Bp // _BM
    qrows = Bp // _NQ
    I = g_ref.shape[0]
    N = g_ref.shape[1]

    def in_copy(q):
        return pltpu.make_async_copy(
            x_hbm.at[pl.ds(q * qrows, qrows)],
            xbuf.at[pl.ds(q * qrows, qrows)],
            insem.at[q])

    def out_copy(i, slot):
        return pltpu.make_async_copy(
            obuf.at[slot], o_hbm.at[pl.ds(i * _BM, _BM)], outsem.at[slot])

    for q in range(_NQ):
        in_copy(q).start()

    steps_per_q = nsteps // _NQ

    def loop(i, carry):
        slot = jax.lax.rem(i, _NSLOT)
        @pl.when(jax.lax.rem(i, steps_per_q) == 0)
        def _():
            in_copy(jax.lax.div(i, steps_per_q)).wait()
        u = jnp.maximum(
            xbuf[pl.ds(i * _BM, _BM)] * cw_ref[...] + cb_ref[...], 0.0)
        g = g_ref[...]
        oute = jnp.dot(u[:, :I], g, preferred_element_type=jnp.float32)
        outo = jnp.dot(u[:, I:], g, preferred_element_type=jnp.float32)
        @pl.when(i >= _NSLOT)
        def _():
            out_copy(i - _NSLOT, slot).wait()
        obuf[slot, :, :N] = jnp.maximum(oute + b2_ref[...], 0.0)
        obuf[slot, :, N:] = jnp.maximum(outo + b2_ref[...], 0.0)
        out_copy(i, slot).start()
        return carry

    jax.lax.fori_loop(0, nsteps, loop, 0)
    for j in range(max(0, nsteps - _NSLOT), nsteps):
        out_copy(j, j % _NSLOT).wait()


def kernel(x, conv_w, conv_b, W0, b0, W1, b1, W2, b2):
    B, I = x.shape
    N = W2.shape[0]
    K = W2.shape[1] // 2
    G = jnp.zeros((I, N), x.dtype).at[33::2, :].set(W2[:, :K].T)
    cw2 = jnp.tile(conv_w, 2)[None]
    cb2 = jnp.tile(conv_b, 2)[None]
    xr = x.reshape(B // 2, 2 * I)
    vmem = pl.BlockSpec(memory_space=pltpu.VMEM)
    hbm = pl.BlockSpec(memory_space=pl.ANY)
    out = pl.pallas_call(
        _body,
        in_specs=[vmem, vmem, vmem, vmem, hbm],
        out_specs=hbm,
        out_shape=jax.ShapeDtypeStruct((B // 2, 2 * N), x.dtype),
        scratch_shapes=[
            pltpu.VMEM((B // 2, 2 * I), x.dtype),
            pltpu.VMEM((_NSLOT, _BM, 2 * N), x.dtype),
            pltpu.SemaphoreType.DMA((_NQ,)),
            pltpu.SemaphoreType.DMA((_NSLOT,)),
        ],
    )(cw2, cb2, G, b2[None], xr)
    return out.reshape(B, N)


# speedup vs baseline: 2.7200x; 2.5851x over previous
import jax
import jax.numpy as jnp
from jax.experimental import pallas as pl
from jax.experimental.pallas import tpu as pltpu

_BM = 1024
_NSLOT = 6
_NQ = 8


def _body(cw_ref, cb_ref, g_ref, b2_ref, x_hbm, o_hbm, xbuf, obuf, insem, outsem):
    B = x_hbm.shape[0]
    nsteps = B // _BM
    qrows = B // _NQ

    def in_copy(q):
        return pltpu.make_async_copy(
            x_hbm.at[pl.ds(q * qrows, qrows)],
            xbuf.at[pl.ds(q * qrows, qrows)],
            insem.at[q])

    def out_copy(i, slot):
        return pltpu.make_async_copy(
            obuf.at[slot], o_hbm.at[pl.ds(i * _BM, _BM)], outsem.at[slot])

    for q in range(_NQ):
        in_copy(q).start()

    steps_per_q = nsteps // _NQ

    def loop(i, carry):
        slot = jax.lax.rem(i, _NSLOT)
        @pl.when(jax.lax.rem(i, steps_per_q) == 0)
        def _():
            in_copy(jax.lax.div(i, steps_per_q)).wait()
        u = jnp.maximum(
            xbuf[pl.ds(i * _BM, _BM)] * cw_ref[...] + cb_ref[...], 0.0)
        acc = jnp.dot(u, g_ref[...], preferred_element_type=jnp.float32)
        @pl.when(i >= _NSLOT)
        def _():
            out_copy(i - _NSLOT, slot).wait()
        obuf[slot] = jnp.maximum(acc + b2_ref[...], 0.0)
        out_copy(i, slot).start()
        return carry

    jax.lax.fori_loop(0, nsteps, loop, 0)
    for j in range(max(0, nsteps - _NSLOT), nsteps):
        out_copy(j, j % _NSLOT).wait()


def kernel(x, conv_w, conv_b, W0, b0, W1, b1, W2, b2):
    B, I = x.shape
    N = W2.shape[0]
    K = W2.shape[1] // 2
    G = jnp.zeros((I, N), x.dtype).at[33::2, :].set(W2[:, :K].T)
    vmem = pl.BlockSpec(memory_space=pltpu.VMEM)
    hbm = pl.BlockSpec(memory_space=pl.ANY)
    out = pl.pallas_call(
        _body,
        in_specs=[vmem, vmem, vmem, vmem, hbm],
        out_specs=hbm,
        out_shape=jax.ShapeDtypeStruct((B, N), x.dtype),
        scratch_shapes=[
            pltpu.VMEM((B, I), x.dtype),
            pltpu.VMEM((_NSLOT, _BM, N), x.dtype),
            pltpu.SemaphoreType.DMA((_NQ,)),
            pltpu.SemaphoreType.DMA((_NSLOT,)),
        ],
    )(conv_w[None], conv_b[None], G, b2[None], x)
    return out
